# baseline (speedup 1.0000x reference)
import jax
import jax.numpy as jnp
from jax import lax
from jax.experimental import pallas as pl
from jax.experimental.pallas import tpu as pltpu

L = 256
NFEAT = 128
J = 128
SPK = 4


def _ggcn1_kernel(x_hbm, pidx_hbm, h1w_hbm, g1w_hbm, fw_hbm, out_ref,
                  x_v, pidx_v, h1w_v, g1w_v, fw_v,
                  sx, spidx, sh, sg, sf):
    cx = pltpu.make_async_copy(x_hbm, x_v, sx)
    cp = pltpu.make_async_copy(pidx_hbm, pidx_v, spidx)
    ch = pltpu.make_async_copy(h1w_hbm, h1w_v, sh)
    cg = pltpu.make_async_copy(g1w_hbm, g1w_v, sg)
    cf = pltpu.make_async_copy(fw_hbm, fw_v, sf)
    cx.start(); cp.start(); ch.start(); cg.start(); cf.start()

    cx.wait(); ch.wait()
    h_all = jnp.maximum(
        jnp.dot(x_v[...], h1w_v[...], preferred_element_type=jnp.float32), 0.0)

    cg.wait()
    p_top = jnp.dot(h_all, g1w_v[:J, :], preferred_element_type=jnp.float32)
    q_bot = jnp.dot(h_all, g1w_v[J:, :], preferred_element_type=jnp.float32)

    def roll_both(m):
        return (jnp.concatenate([m[L - 1:, :], m[:L - 1, :]], axis=0),
                jnp.concatenate([m[1:, :], m[:1, :]], axis=0))

    p_m1, p_p1 = roll_both(p_top)
    q_m1, q_p1 = roll_both(q_bot)

    cp.wait()
    iota = lax.broadcasted_iota(jnp.int32, (L, 1), 0)
    pidx = pidx_v[...]
    is_m1 = pidx == jnp.where(iota == 0, L - 1, iota - 1)
    is_p1 = pidx == jnp.where(iota == L - 1, 0, iota + 1)

    def sel(col, m_m1, m_p1, m_0):
        mm = is_m1[:, col:col + 1]
        mp = is_p1[:, col:col + 1]
        return jnp.where(mm, m_m1, jnp.where(mp, m_p1, m_0))

    acc = jnp.zeros((L, J), dtype=jnp.float32)
    for s in range(SPK):
        a = sel(0 * SPK + s, p_m1, p_p1, p_top)
        b = sel(1 * SPK + s, q_m1, q_p1, q_bot)
        acc = acc + jnp.maximum(a + b, 0.0)

    e = acc * (1.0 / SPK)
    e2 = jnp.maximum(
        p_top + jnp.dot(e, g1w_v[J:, :], preferred_element_type=jnp.float32),
        0.0)
    cf.wait()
    out_ref[...] = jnp.dot(e2, fw_v[...], preferred_element_type=jnp.float32)


def kernel(X_, perm_idx, h1_w, h1_b, g1_w, g1_b, f_w, f_b):
    pidx2d = jnp.reshape(perm_idx, (L, 2 * SPK))
    return pl.pallas_call(
        _ggcn1_kernel,
        out_shape=jax.ShapeDtypeStruct((L, 1), jnp.float32),
        in_specs=[pl.BlockSpec(memory_space=pl.ANY)] * 5,
        scratch_shapes=[
            pltpu.VMEM((L, NFEAT), jnp.float32),
            pltpu.VMEM((L, 2 * SPK), jnp.int32),
            pltpu.VMEM((NFEAT, J), jnp.float32),
            pltpu.VMEM((2 * J, J), jnp.float32),
            pltpu.VMEM((J, 1), jnp.float32),
            pltpu.SemaphoreType.DMA,
            pltpu.SemaphoreType.DMA,
            pltpu.SemaphoreType.DMA,
            pltpu.SemaphoreType.DMA,
            pltpu.SemaphoreType.DMA,
        ],
    )(X_, pidx2d, h1_w, g1_w, f_w)
